# baseline (speedup 1.0000x reference)
import functools
from functools import partial

import jax
import jax.numpy as jnp
from jax import lax
from jax.experimental import pallas as pl
from jax.experimental.pallas import tpu as pltpu
from jax.experimental.pallas import tpu_sc as plsc

K = 16
G = 32

NC, NS = 2, 16
NW = NC * NS
_P = 2048


def _cdiv(a, b):
    return (a + b - 1) // b


def _padn(m):
    return _cdiv(m, _P) * _P


def _pad_rows(x, rows, fill=0):
    return jnp.pad(x, ((0, rows - x.shape[0]),) + ((0, 0),) * (x.ndim - 1),
                   constant_values=fill)


def _pick_bc(rows_pw, cap=128):
    cap = max(8, min(128, cap))
    for bc in range(cap - cap % 8, 7, -8):
        if rows_pw % bc == 0:
            return bc
    return 8


def _sc_mesh():
    return plsc.VectorSubcoreMesh(core_axis_name="c", subcore_axis_name="s",
                                  num_cores=NC, num_subcores=NS)


def sc_gather(src, idx_list):
    S, C = src.shape
    nidx = len(idx_list)
    Fpad = idx_list[0].shape[0]
    rows_pw = Fpad // NW
    BC = _pick_bc(rows_pw, 100000 // (2 * nidx * C))
    nchunks = rows_pw // BC
    scratch = ([pltpu.VMEM((rows_pw,), jnp.int32)] * nidx
               + [pltpu.VMEM((BC, C), jnp.float32)] * (2 * nidx)
               + [pltpu.SemaphoreType.DMA, pltpu.SemaphoreType.DMA])

    def body(*args):
        src_hbm = args[0]
        idx_hbm = args[1:1 + nidx]
        out_hbm = args[1 + nidx]
        idx_v = args[2 + nidx:2 + 2 * nidx]
        bufs = [args[2 + 2 * nidx:2 + 3 * nidx],
                args[2 + 3 * nidx:2 + 4 * nidx]]
        gsem, osem = args[-2], args[-1]
        wid = lax.axis_index("s") * NC + lax.axis_index("c")
        base = wid * rows_pw
        for j in range(nidx):
            pltpu.sync_copy(idx_hbm[j].at[pl.ds(base, rows_pw)], idx_v[j])

        def fire_gather(jc):
            off = jc * BC
            sl = bufs[jc % 2]
            return [pltpu.async_copy(
                src_hbm.at[idx_v[j].at[pl.ds(off, BC)]], sl[j], gsem)
                for j in range(nidx)]

        def fire_out(jc):
            off = jc * BC
            sl = bufs[jc % 2]
            return [pltpu.async_copy(
                sl[j], out_hbm.at[j, pl.ds(base + off, BC)], osem)
                for j in range(nidx)]

        gd = {0: fire_gather(0)}
        od = {}
        for jc in range(nchunks):
            if jc + 1 < nchunks:
                if jc >= 1:
                    for d in od.pop(jc - 1):
                        d.wait()
                gd[jc + 1] = fire_gather(jc + 1)
            for d in gd.pop(jc):
                d.wait()
            od[jc] = fire_out(jc)
        for descs in od.values():
            for d in descs:
                d.wait()

    return pl.kernel(
        body,
        out_type=jax.ShapeDtypeStruct((nidx, Fpad, C), jnp.float32),
        mesh=_sc_mesh(),
        scratch_types=scratch,
        compiler_params=pltpu.CompilerParams(use_tc_tiling_on_sc=False),
        name=f"scg{nidx}_c{C}_f{Fpad}",
    )(src, *idx_list)


def _red_body(op, x_ref, out_ref):
    x = x_ref[...]
    acc = x[0]
    for j in range(1, x.shape[0]):
        acc = acc + x[j] if op == 'sum' else jnp.maximum(acc, x[j])
    out_ref[...] = acc


def comb_op(x, op, bm=2048):
    nidx, npad, C = x.shape
    return pl.pallas_call(
        partial(_red_body, op),
        grid=(_cdiv(npad, bm),),
        in_specs=[pl.BlockSpec((nidx, bm, C), lambda i: (0, i, 0))],
        out_specs=pl.BlockSpec((bm, C), lambda i: (i, 0)),
        out_shape=jax.ShapeDtypeStruct((npad, C), jnp.float32),
    )(x)


def sc_scatter_add3(msg, i0, i1, i2, npad):
    Fpad, GC = msg.shape
    rows_pw = Fpad // NW
    BC = _pick_bc(rows_pw)
    nchunks = rows_pw // BC
    i0r = i0.reshape(Fpad // BC, BC)
    i1r = i1.reshape(Fpad // BC, BC)
    i2r = i2.reshape(Fpad // BC, BC)
    BCZ = 128
    zrows = npad // NS
    nz = zrows // BCZ
    scratch = ([pltpu.VMEM((nchunks, BC), jnp.int32)] * 3
               + [pltpu.VMEM((BC, GC), jnp.float32)] * 2
               + [pltpu.VMEM((BCZ, GC), jnp.float32),
                  pltpu.VMEM_SHARED((npad, GC), jnp.float32),
                  pltpu.SemaphoreType.DMA, pltpu.SemaphoreType.DMA])

    def body(msg_hbm, i0h, i1h, i2h, out_hbm, iv0, iv1, iv2, mb0, mb1, zbuf,
             shared, msem, ssem):
        cid = lax.axis_index("c")
        sid = lax.axis_index("s")
        wid = sid * NC + cid
        mbufs = (mb0, mb1)

        def zrow(r, c2):
            for c0 in range(GC // 16):
                zbuf[r, pl.ds(c0 * 16, 16)] = jnp.zeros((16,), jnp.float32)
            return c2
        lax.fori_loop(0, BCZ, zrow, 0)
        zd = [pltpu.async_copy(
            zbuf, shared.at[pl.ds(sid * zrows + j * BCZ, BCZ)], msem)
            for j in range(nz)]
        for d in zd:
            d.wait()
        plsc.subcore_barrier()

        chunk0 = wid * nchunks
        pltpu.sync_copy(i0h.at[pl.ds(chunk0, nchunks)], iv0)
        pltpu.sync_copy(i1h.at[pl.ds(chunk0, nchunks)], iv1)
        pltpu.sync_copy(i2h.at[pl.ds(chunk0, nchunks)], iv2)

        def fire_msg(jc):
            return pltpu.async_copy(
                msg_hbm.at[pl.ds((chunk0 + jc) * BC, BC)], mbufs[jc % 2], msem)

        def fire_scat(jc):
            return [pltpu.async_copy(mbufs[jc % 2], shared.at[iv.at[jc]],
                                     ssem, add=True)
                    for iv in (iv0, iv1, iv2)]

        md = {0: fire_msg(0)}
        sd = {}
        for jc in range(nchunks):
            if jc + 1 < nchunks:
                if jc >= 1:
                    for d in sd.pop(jc - 1):
                        d.wait()
                md[jc + 1] = fire_msg(jc + 1)
            md.pop(jc).wait()
            sd[jc] = fire_scat(jc)
        for descs in sd.values():
            for d in descs:
                d.wait()
        plsc.subcore_barrier()

        od = [pltpu.async_copy(
            shared.at[pl.ds(sid * zrows + j * BCZ, BCZ)],
            out_hbm.at[cid, pl.ds(sid * zrows + j * BCZ, BCZ)], msem)
            for j in range(nz)]
        for d in od:
            d.wait()

    return pl.kernel(
        body,
        out_type=jax.ShapeDtypeStruct((2, npad, GC), jnp.float32),
        mesh=_sc_mesh(),
        scratch_types=scratch,
        compiler_params=pltpu.CompilerParams(use_tc_tiling_on_sc=False),
        name=f"scs_g{GC}_f{Fpad}_n{npad}",
    )(msg, i0r, i1r, i2r)


def _comb_body(relu, p0_ref, p1_ref, inv_ref, out_ref):
    acc = (p0_ref[0] + p1_ref[0]) * inv_ref[...]
    if relu:
        acc = jnp.maximum(acc, 0.0)
    out_ref[...] = acc


def combine_partials(p, inv_nf, relu, bm=1024):
    _, npad, GC = p.shape
    return pl.pallas_call(
        partial(_comb_body, relu),
        grid=(_cdiv(npad, bm),),
        in_specs=[
            pl.BlockSpec((1, bm, GC), lambda i: (0, i, 0)),
            pl.BlockSpec((1, bm, GC), lambda i: (1, i, 0)),
            pl.BlockSpec((bm, GC), lambda i: (i, 0)),
        ],
        out_specs=pl.BlockSpec((bm, GC), lambda i: (i, 0)),
        out_shape=jax.ShapeDtypeStruct((npad, GC), jnp.float32),
    )(p, p, inv_nf)


def _nf_body(p0_ref, p1_ref, out_ref):
    s = p0_ref[0, :, 0:1] + p1_ref[0, :, 0:1]
    s = jnp.maximum(s, 1.0)
    out_ref[...] = jnp.broadcast_to(1.0 / s, out_ref.shape)


def nf_inverse(p, bm=1024):
    _, npad, GC = p.shape
    return pl.pallas_call(
        _nf_body,
        grid=(_cdiv(npad, bm),),
        in_specs=[
            pl.BlockSpec((1, bm, GC), lambda i: (0, i, 0)),
            pl.BlockSpec((1, bm, GC), lambda i: (1, i, 0)),
        ],
        out_specs=pl.BlockSpec((bm, G), lambda i: (i, 0)),
        out_shape=jax.ShapeDtypeStruct((npad, G), jnp.float32),
    )(p, p)


def _conv_body(nrows, bf, ff_ref, coeff_ref, wp_ref, b_ref, out_ref):
    ff = ff_ref[...]
    if ff.ndim == 3:
        ff = ff[0] + ff[1] + ff[2]
    t = jnp.dot(ff, wp_ref[...], preferred_element_type=jnp.float32)
    coeff = coeff_ref[...]
    acc = b_ref[0, :][None, :] + coeff[:, 0:1] * t[:, 0:G]
    for k in range(1, K):
        acc = acc + coeff[:, k:k + 1] * t[:, k * G:(k + 1) * G]
    rid = pl.program_id(0) * bf + jax.lax.broadcasted_iota(jnp.int32, acc.shape, 0)
    out_ref[...] = jnp.where(rid < nrows, acc, 0.0)


def conv_matmul(ff, coeff, Wp, b, nrows, bf=512):
    if ff.ndim == 3:
        _, F, cin = ff.shape
        ff_spec = pl.BlockSpec((3, bf, cin), lambda i: (0, i, 0))
    else:
        F, cin = ff.shape
        ff_spec = pl.BlockSpec((bf, cin), lambda i: (i, 0))
    grid = (_cdiv(F, bf),)
    return pl.pallas_call(
        partial(_conv_body, nrows, bf),
        grid=grid,
        in_specs=[
            ff_spec,
            pl.BlockSpec((bf, K), lambda i: (i, 0)),
            pl.BlockSpec((cin, K * G), lambda i: (0, 0)),
            pl.BlockSpec((1, G), lambda i: (0, 0)),
        ],
        out_specs=pl.BlockSpec((bf, G), lambda i: (i, 0)),
        out_shape=jax.ShapeDtypeStruct((F, G), jnp.float32),
    )(ff, coeff, Wp, b.reshape(1, G))


def _mm_body(relu, x_ref, w_ref, b_ref, out_ref):
    acc = jnp.dot(x_ref[...], w_ref[...], preferred_element_type=jnp.float32)
    acc = acc + b_ref[0, :][None, :]
    if relu:
        acc = jnp.maximum(acc, 0.0)
    out_ref[...] = acc


def dense(x, W, b, relu=True, bm=512):
    M, Kd = x.shape
    N = W.shape[1]
    return pl.pallas_call(
        partial(_mm_body, relu),
        grid=(_cdiv(M, bm),),
        in_specs=[
            pl.BlockSpec((bm, Kd), lambda i: (i, 0)),
            pl.BlockSpec((Kd, N), lambda i: (0, 0)),
            pl.BlockSpec((1, N), lambda i: (0, 0)),
        ],
        out_specs=pl.BlockSpec((bm, N), lambda i: (i, 0)),
        out_shape=jax.ShapeDtypeStruct((M, N), jnp.float32),
    )(x, W, b.reshape(1, N))


def _pool_bounds(n, nn):
    t = jnp.arange(nn, dtype=jnp.int32)
    a = (t * n + nn - 1) // nn
    b = ((t + 1) * n + nn - 1) // nn - 1
    return a, b


def _sph_harm_coeff(n):
    x = n[:, 0]; y = n[:, 1]; z = n[:, 2]
    x2 = x * x; y2 = y * y; z2 = z * z
    c = [
        0.28209479177387814 * jnp.ones_like(x),
        0.4886025119029199 * y,
        0.4886025119029199 * z,
        0.4886025119029199 * x,
        1.0925484305920792 * x * y,
        1.0925484305920792 * y * z,
        0.31539156525252005 * (3.0 * z2 - 1.0),
        1.0925484305920792 * x * z,
        0.5462742152960396 * (x2 - y2),
        0.5900435899266435 * y * (3.0 * x2 - y2),
        2.890611442640554 * x * y * z,
        0.4570457994644658 * y * (5.0 * z2 - 1.0),
        0.3731763325901154 * z * (5.0 * z2 - 3.0),
        0.4570457994644658 * x * (5.0 * z2 - 1.0),
        1.445305721320277 * z * (x2 - y2),
        0.5900435899266435 * x * (x2 - 3.0 * y2),
    ]
    return jnp.stack(c, axis=1)


def _snorm(v):
    return jnp.sqrt(jnp.sum(v * v, axis=-1, keepdims=True) + 1e-12)


def _face_geom(V1, V2, V3, with_geo):
    nrm = jnp.cross(V2 - V1, V3 - V1)
    nrm = nrm / _snorm(nrm)
    coeff = _sph_harm_coeff(nrm)
    if not with_geo:
        return coeff, None
    D12 = V2 - V1; D23 = V3 - V2; D31 = V1 - V3
    L12 = _snorm(D12); L23 = _snorm(D23); L31 = _snorm(D31)
    eps = 1e-8
    T1 = jnp.sum(D12 * -D31, axis=-1, keepdims=True) / (L12 * L31 + eps)
    T2 = jnp.sum(-D12 * D23, axis=-1, keepdims=True) / (L12 * L23 + eps)
    T3 = jnp.sum(-D23 * D31, axis=-1, keepdims=True) / (L23 * L31 + eps)
    geo = jnp.concatenate([L12, L23, L31, T1, T2, T3, nrm], axis=-1)
    return coeff, geo


def kernel(vertex_in, face_in, nv_in, mf_in, params):
    verts = vertex_in[:, :3]

    ns = [verts.shape[0]]
    Fs = [face_in.shape[0]]
    levels_f = [face_in]
    vt_infos = []
    n = ns[0]
    f = face_in
    for k in range(4):
        nn = int(n / 1.5)
        a, b = _pool_bounds(n, nn)
        vt_infos.append((n, nn, a, b))
        fn = int(f.shape[0] / 1.5)
        f = ((f * nn) // n)[:fn]
        levels_f.append(f)
        Fs.append(fn)
        n = nn
        ns.append(n)
    npads = [_padn(m) for m in ns]
    Fpads = [_padn(m) for m in Fs]

    fidx = [tuple(_pad_rows(levels_f[k][:, j], Fpads[k]) for j in range(3))
            for k in range(5)]

    vs = [_pad_rows(jnp.pad(verts, ((0, 0), (0, 13))), npads[0])]
    for k in range(4):
        n, nn, a, b = vt_infos[k]
        i = jnp.arange(n, dtype=jnp.int32)
        t = (i * nn) // n
        cnt_src = (((t + 1) * n + nn - 1) // nn - (t * n + nn - 1) // nn
                   ).astype(jnp.float32)
        vsrc = _pad_rows(vs[k][:n] * (1.0 / cnt_src)[:, None], npads[k])
        apad = _pad_rows(a, npads[k + 1])
        bpad = _pad_rows(jnp.where(b > a, b, n), npads[k + 1])
        vs.append(comb_op(sc_gather(vsrc, [apad, bpad]), 'sum'))

    coeffs = []
    inv_nfs = []
    geo0 = None
    for k in range(5):
        g3 = sc_gather(vs[k], list(fidx[k]))
        V1 = g3[0][:, :3]
        V2 = g3[1][:, :3]
        V3 = g3[2][:, :3]
        coeff, geo = _face_geom(V1, V2, V3, with_geo=(k == 0))
        if k == 0:
            geo0 = jnp.pad(geo, ((0, 0), (0, 16 - 9)))
        coeffs.append(coeff)
        ones = _pad_rows(jnp.ones((Fs[k], 16), jnp.float32), Fpads[k])
        nfp = sc_scatter_add3(ones, *fidx[k], npads[k])
        inv_nfs.append(nf_inverse(nfp))

    def prep_W(W):
        cin = W.shape[1]
        return W.transpose(1, 0, 2).reshape(cin, K * G)

    def conv_layer(ff, k, W, b, third=True):
        Wp = prep_W(W)
        if third:
            Wp = Wp / 3.0
        msg = conv_matmul(ff, coeffs[k], Wp, b, Fs[k])
        p = sc_scatter_add3(msg, *fidx[k], npads[k])
        return combine_partials(p, inv_nfs[k], relu=True)

    W0 = jnp.pad(params['conv0_W'], ((0, 0), (0, 16 - 9), (0, 0)))
    feats = conv_layer(geo0, 0, W0, params['conv0_b'], third=False)

    skips = []
    for k in range(5):
        bp = params['blocks'][k]
        ff = None
        new = feats
        for W, b in zip(bp['Ws'], bp['bs']):
            ffnew = sc_gather(new, list(fidx[k]))
            ff = ffnew if ff is None else jnp.concatenate([ff, ffnew], axis=2)
            new = conv_layer(ff, k, W, b)
            feats = jnp.concatenate([feats, new], axis=1)
        feats = dense(feats, bp['Wout'], bp['bout'], relu=True)
        if k < 4:
            skips.append(feats)
            n, nn, a, b = vt_infos[k]
            apad = _pad_rows(a, npads[k + 1])
            bpad = _pad_rows(b, npads[k + 1])
            feats = comb_op(sc_gather(feats, [apad, bpad]), 'max')

    for k in range(4):
        it = 4 - k
        n, nn, a, b = vt_infos[it - 1]
        vt_map = (jnp.arange(n, dtype=jnp.int32) * nn) // n
        up = sc_gather(feats, [_pad_rows(vt_map, npads[it - 1])])[0]
        feats = jnp.concatenate([skips[it - 1], up], axis=1)
        W, b = params['dec'][k]
        feats = dense(feats, W, b, relu=True)

    predW = jnp.pad(params['pred_W'], ((0, 0), (0, 128 - 13)))
    predb = jnp.pad(params['pred_b'], (0, 128 - 13))
    out = dense(feats, predW, predb, relu=False)
    return out[:ns[0], :13]


# speedup vs baseline: 1.1296x; 1.1296x over previous
import functools
from functools import partial

import jax
import jax.numpy as jnp
from jax import lax
from jax.experimental import pallas as pl
from jax.experimental.pallas import tpu as pltpu
from jax.experimental.pallas import tpu_sc as plsc

K = 16
G = 32

NC, NS = 2, 16
NW = NC * NS
_P = 2048


def _cdiv(a, b):
    return (a + b - 1) // b


def _padn(m):
    return _cdiv(m, _P) * _P


def _pad_rows(x, rows, fill=0):
    return jnp.pad(x, ((0, rows - x.shape[0]),) + ((0, 0),) * (x.ndim - 1),
                   constant_values=fill)


def _pick_bc(rows_pw, cap=128):
    cap = max(8, min(128, cap))
    for bc in range(cap - cap % 8, 7, -8):
        if rows_pw % bc == 0:
            return bc
    return 8


def _sc_mesh():
    return plsc.VectorSubcoreMesh(core_axis_name="c", subcore_axis_name="s",
                                  num_cores=NC, num_subcores=NS)


def sc_gather(src, idx_list):
    S, C = src.shape
    nidx = len(idx_list)
    Fpad = idx_list[0].shape[0]
    rows_pw = Fpad // NW
    BC = _pick_bc(rows_pw, 100000 // (2 * nidx * C))
    nchunks = rows_pw // BC
    scratch = ([pltpu.VMEM((rows_pw,), jnp.int32)] * nidx
               + [pltpu.VMEM((BC, C), jnp.float32)] * (2 * nidx)
               + [pltpu.SemaphoreType.DMA, pltpu.SemaphoreType.DMA])

    def body(*args):
        src_hbm = args[0]
        idx_hbm = args[1:1 + nidx]
        out_hbm = args[1 + nidx]
        idx_v = args[2 + nidx:2 + 2 * nidx]
        bufs = [args[2 + 2 * nidx:2 + 3 * nidx],
                args[2 + 3 * nidx:2 + 4 * nidx]]
        gsem, osem = args[-2], args[-1]
        wid = lax.axis_index("s") * NC + lax.axis_index("c")
        base = wid * rows_pw
        for j in range(nidx):
            pltpu.sync_copy(idx_hbm[j].at[pl.ds(base, rows_pw)], idx_v[j])

        def fire_gather(jc):
            off = jc * BC
            sl = bufs[jc % 2]
            return [pltpu.async_copy(
                src_hbm.at[idx_v[j].at[pl.ds(off, BC)]], sl[j], gsem)
                for j in range(nidx)]

        def fire_out(jc):
            off = jc * BC
            sl = bufs[jc % 2]
            return [pltpu.async_copy(
                sl[j], out_hbm.at[j, pl.ds(base + off, BC)], osem)
                for j in range(nidx)]

        gd = {0: fire_gather(0)}
        od = {}
        for jc in range(nchunks):
            if jc + 1 < nchunks:
                if jc >= 1:
                    for d in od.pop(jc - 1):
                        d.wait()
                gd[jc + 1] = fire_gather(jc + 1)
            for d in gd.pop(jc):
                d.wait()
            od[jc] = fire_out(jc)
        for descs in od.values():
            for d in descs:
                d.wait()

    return pl.kernel(
        body,
        out_type=jax.ShapeDtypeStruct((nidx, Fpad, C), jnp.float32),
        mesh=_sc_mesh(),
        scratch_types=scratch,
        compiler_params=pltpu.CompilerParams(use_tc_tiling_on_sc=False),
        name=f"scg{nidx}_c{C}_f{Fpad}",
    )(src, *idx_list)


def sc_gather_comb(src, idx_list, op):
    S, C = src.shape
    nidx = len(idx_list)
    Fpad = idx_list[0].shape[0]
    rows_pw = Fpad // NW
    BC = _pick_bc(rows_pw, 100000 // (nidx * C))
    nchunks = rows_pw // BC
    scratch = ([pltpu.VMEM((rows_pw,), jnp.int32)] * nidx
               + [pltpu.VMEM((BC, C), jnp.float32)] * nidx
               + [pltpu.SemaphoreType.DMA])

    def body(*args):
        src_hbm = args[0]
        idx_hbm = args[1:1 + nidx]
        out_hbm = args[1 + nidx]
        idx_v = args[2 + nidx:2 + 2 * nidx]
        bufs = args[2 + 2 * nidx:2 + 3 * nidx]
        sem = args[-1]
        wid = lax.axis_index("s") * NC + lax.axis_index("c")
        base = wid * rows_pw
        for j in range(nidx):
            pltpu.sync_copy(idx_hbm[j].at[pl.ds(base, rows_pw)], idx_v[j])

        def chunk(jc, carry):
            off = jc * BC
            cps = [pltpu.async_copy(src_hbm.at[idx_v[j].at[pl.ds(off, BC)]],
                                    bufs[j], sem) for j in range(nidx)]
            for cp in cps:
                cp.wait()

            def row(r, c2):
                for c0 in range(C // 16):
                    sl = pl.ds(c0 * 16, 16)
                    if op == 'sum':
                        acc = bufs[0][r, sl] + bufs[1][r, sl]
                        if nidx == 3:
                            acc = acc + bufs[2][r, sl]
                        bufs[0][r, sl] = acc
                    else:
                        bufs[0][r, sl] = jnp.maximum(bufs[0][r, sl],
                                                     bufs[1][r, sl])
                return c2
            lax.fori_loop(0, BC, row, 0)
            pltpu.sync_copy(bufs[0], out_hbm.at[pl.ds(base + off, BC)])
            return carry
        lax.fori_loop(0, nchunks, chunk, 0)

    return pl.kernel(
        body,
        out_type=jax.ShapeDtypeStruct((Fpad, C), jnp.float32),
        mesh=_sc_mesh(),
        scratch_types=scratch,
        compiler_params=pltpu.CompilerParams(use_tc_tiling_on_sc=False),
        name=f"scgc_{op}{nidx}_c{C}_f{Fpad}",
    )(src, *idx_list)


def _red_body(op, x_ref, out_ref):
    x = x_ref[...]
    acc = x[0]
    for j in range(1, x.shape[0]):
        acc = acc + x[j] if op == 'sum' else jnp.maximum(acc, x[j])
    out_ref[...] = acc


def comb_op(x, op, bm=2048):
    nidx, npad, C = x.shape
    return pl.pallas_call(
        partial(_red_body, op),
        grid=(_cdiv(npad, bm),),
        in_specs=[pl.BlockSpec((nidx, bm, C), lambda i: (0, i, 0))],
        out_specs=pl.BlockSpec((bm, C), lambda i: (i, 0)),
        out_shape=jax.ShapeDtypeStruct((npad, C), jnp.float32),
    )(x)


def sc_scatter_add3(msg, i0, i1, i2, npad):
    Fpad, GC = msg.shape
    rows_pw = Fpad // NW
    BC = _pick_bc(rows_pw)
    nchunks = rows_pw // BC
    i0r = i0.reshape(Fpad // BC, BC)
    i1r = i1.reshape(Fpad // BC, BC)
    i2r = i2.reshape(Fpad // BC, BC)
    BCZ = 128
    zrows = npad // NS
    nz = zrows // BCZ
    scratch = ([pltpu.VMEM((nchunks, BC), jnp.int32)] * 3
               + [pltpu.VMEM((BC, GC), jnp.float32)] * 2
               + [pltpu.VMEM((BCZ, GC), jnp.float32),
                  pltpu.VMEM_SHARED((npad, GC), jnp.float32),
                  pltpu.SemaphoreType.DMA, pltpu.SemaphoreType.DMA])

    def body(msg_hbm, i0h, i1h, i2h, out_hbm, iv0, iv1, iv2, mb0, mb1, zbuf,
             shared, msem, ssem):
        cid = lax.axis_index("c")
        sid = lax.axis_index("s")
        wid = sid * NC + cid
        mbufs = (mb0, mb1)

        def zrow(r, c2):
            for c0 in range(GC // 16):
                zbuf[r, pl.ds(c0 * 16, 16)] = jnp.zeros((16,), jnp.float32)
            return c2
        lax.fori_loop(0, BCZ, zrow, 0)
        zd = [pltpu.async_copy(
            zbuf, shared.at[pl.ds(sid * zrows + j * BCZ, BCZ)], msem)
            for j in range(nz)]
        for d in zd:
            d.wait()
        plsc.subcore_barrier()

        chunk0 = wid * nchunks
        pltpu.sync_copy(i0h.at[pl.ds(chunk0, nchunks)], iv0)
        pltpu.sync_copy(i1h.at[pl.ds(chunk0, nchunks)], iv1)
        pltpu.sync_copy(i2h.at[pl.ds(chunk0, nchunks)], iv2)

        def fire_msg(jc):
            return pltpu.async_copy(
                msg_hbm.at[pl.ds((chunk0 + jc) * BC, BC)], mbufs[jc % 2], msem)

        def fire_scat(jc):
            return [pltpu.async_copy(mbufs[jc % 2], shared.at[iv.at[jc]],
                                     ssem, add=True)
                    for iv in (iv0, iv1, iv2)]

        md = {0: fire_msg(0)}
        sd = {}
        for jc in range(nchunks):
            if jc + 1 < nchunks:
                if jc >= 1:
                    for d in sd.pop(jc - 1):
                        d.wait()
                md[jc + 1] = fire_msg(jc + 1)
            md.pop(jc).wait()
            sd[jc] = fire_scat(jc)
        for descs in sd.values():
            for d in descs:
                d.wait()
        plsc.subcore_barrier()

        od = [pltpu.async_copy(
            shared.at[pl.ds(sid * zrows + j * BCZ, BCZ)],
            out_hbm.at[cid, pl.ds(sid * zrows + j * BCZ, BCZ)], msem)
            for j in range(nz)]
        for d in od:
            d.wait()

    return pl.kernel(
        body,
        out_type=jax.ShapeDtypeStruct((2, npad, GC), jnp.float32),
        mesh=_sc_mesh(),
        scratch_types=scratch,
        compiler_params=pltpu.CompilerParams(use_tc_tiling_on_sc=False),
        name=f"scs_g{GC}_f{Fpad}_n{npad}",
    )(msg, i0r, i1r, i2r)


def _comb_body(relu, p0_ref, p1_ref, inv_ref, out_ref):
    acc = (p0_ref[0] + p1_ref[0]) * inv_ref[...]
    if relu:
        acc = jnp.maximum(acc, 0.0)
    out_ref[...] = acc


def combine_partials(p, inv_nf, relu, bm=1024):
    _, npad, GC = p.shape
    return pl.pallas_call(
        partial(_comb_body, relu),
        grid=(_cdiv(npad, bm),),
        in_specs=[
            pl.BlockSpec((1, bm, GC), lambda i: (0, i, 0)),
            pl.BlockSpec((1, bm, GC), lambda i: (1, i, 0)),
            pl.BlockSpec((bm, GC), lambda i: (i, 0)),
        ],
        out_specs=pl.BlockSpec((bm, GC), lambda i: (i, 0)),
        out_shape=jax.ShapeDtypeStruct((npad, GC), jnp.float32),
    )(p, p, inv_nf)


def _nf_body(p0_ref, p1_ref, out_ref):
    s = p0_ref[0, :, 0:1] + p1_ref[0, :, 0:1]
    s = jnp.maximum(s, 1.0)
    out_ref[...] = jnp.broadcast_to(1.0 / s, out_ref.shape)


def nf_inverse(p, bm=1024):
    _, npad, GC = p.shape
    return pl.pallas_call(
        _nf_body,
        grid=(_cdiv(npad, bm),),
        in_specs=[
            pl.BlockSpec((1, bm, GC), lambda i: (0, i, 0)),
            pl.BlockSpec((1, bm, GC), lambda i: (1, i, 0)),
        ],
        out_specs=pl.BlockSpec((bm, G), lambda i: (i, 0)),
        out_shape=jax.ShapeDtypeStruct((npad, G), jnp.float32),
    )(p, p)


def _conv_body(nrows, bf, ff_ref, coeff_ref, wp_ref, b_ref, out_ref):
    ff = ff_ref[...]
    if ff.ndim == 3:
        ff = ff[0] + ff[1] + ff[2]
    t = jnp.dot(ff, wp_ref[...], preferred_element_type=jnp.float32)
    coeff = coeff_ref[...]
    acc = b_ref[0, :][None, :] + coeff[:, 0:1] * t[:, 0:G]
    for k in range(1, K):
        acc = acc + coeff[:, k:k + 1] * t[:, k * G:(k + 1) * G]
    rid = pl.program_id(0) * bf + jax.lax.broadcasted_iota(jnp.int32, acc.shape, 0)
    out_ref[...] = jnp.where(rid < nrows, acc, 0.0)


def conv_matmul(ff, coeff, Wp, b, nrows, bf=512):
    if ff.ndim == 3:
        _, F, cin = ff.shape
        ff_spec = pl.BlockSpec((3, bf, cin), lambda i: (0, i, 0))
    else:
        F, cin = ff.shape
        ff_spec = pl.BlockSpec((bf, cin), lambda i: (i, 0))
    grid = (_cdiv(F, bf),)
    return pl.pallas_call(
        partial(_conv_body, nrows, bf),
        grid=grid,
        in_specs=[
            ff_spec,
            pl.BlockSpec((bf, K), lambda i: (i, 0)),
            pl.BlockSpec((cin, K * G), lambda i: (0, 0)),
            pl.BlockSpec((1, G), lambda i: (0, 0)),
        ],
        out_specs=pl.BlockSpec((bf, G), lambda i: (i, 0)),
        out_shape=jax.ShapeDtypeStruct((F, G), jnp.float32),
    )(ff, coeff, Wp, b.reshape(1, G))


def _mm_body(relu, x_ref, w_ref, b_ref, out_ref):
    acc = jnp.dot(x_ref[...], w_ref[...], preferred_element_type=jnp.float32)
    acc = acc + b_ref[0, :][None, :]
    if relu:
        acc = jnp.maximum(acc, 0.0)
    out_ref[...] = acc


def dense(x, W, b, relu=True, bm=512):
    M, Kd = x.shape
    N = W.shape[1]
    return pl.pallas_call(
        partial(_mm_body, relu),
        grid=(_cdiv(M, bm),),
        in_specs=[
            pl.BlockSpec((bm, Kd), lambda i: (i, 0)),
            pl.BlockSpec((Kd, N), lambda i: (0, 0)),
            pl.BlockSpec((1, N), lambda i: (0, 0)),
        ],
        out_specs=pl.BlockSpec((bm, N), lambda i: (i, 0)),
        out_shape=jax.ShapeDtypeStruct((M, N), jnp.float32),
    )(x, W, b.reshape(1, N))


def _pool_bounds(n, nn):
    t = jnp.arange(nn, dtype=jnp.int32)
    a = (t * n + nn - 1) // nn
    b = ((t + 1) * n + nn - 1) // nn - 1
    return a, b


def _sph_harm_coeff(n):
    x = n[:, 0]; y = n[:, 1]; z = n[:, 2]
    x2 = x * x; y2 = y * y; z2 = z * z
    c = [
        0.28209479177387814 * jnp.ones_like(x),
        0.4886025119029199 * y,
        0.4886025119029199 * z,
        0.4886025119029199 * x,
        1.0925484305920792 * x * y,
        1.0925484305920792 * y * z,
        0.31539156525252005 * (3.0 * z2 - 1.0),
        1.0925484305920792 * x * z,
        0.5462742152960396 * (x2 - y2),
        0.5900435899266435 * y * (3.0 * x2 - y2),
        2.890611442640554 * x * y * z,
        0.4570457994644658 * y * (5.0 * z2 - 1.0),
        0.3731763325901154 * z * (5.0 * z2 - 3.0),
        0.4570457994644658 * x * (5.0 * z2 - 1.0),
        1.445305721320277 * z * (x2 - y2),
        0.5900435899266435 * x * (x2 - 3.0 * y2),
    ]
    return jnp.stack(c, axis=1)


def _snorm(v):
    return jnp.sqrt(jnp.sum(v * v, axis=-1, keepdims=True) + 1e-12)


def _face_geom(V1, V2, V3, with_geo):
    nrm = jnp.cross(V2 - V1, V3 - V1)
    nrm = nrm / _snorm(nrm)
    coeff = _sph_harm_coeff(nrm)
    if not with_geo:
        return coeff, None
    D12 = V2 - V1; D23 = V3 - V2; D31 = V1 - V3
    L12 = _snorm(D12); L23 = _snorm(D23); L31 = _snorm(D31)
    eps = 1e-8
    T1 = jnp.sum(D12 * -D31, axis=-1, keepdims=True) / (L12 * L31 + eps)
    T2 = jnp.sum(-D12 * D23, axis=-1, keepdims=True) / (L12 * L23 + eps)
    T3 = jnp.sum(-D23 * D31, axis=-1, keepdims=True) / (L23 * L31 + eps)
    geo = jnp.concatenate([L12, L23, L31, T1, T2, T3, nrm], axis=-1)
    return coeff, geo


def kernel(vertex_in, face_in, nv_in, mf_in, params):
    verts = vertex_in[:, :3]

    ns = [verts.shape[0]]
    Fs = [face_in.shape[0]]
    levels_f = [face_in]
    vt_infos = []
    n = ns[0]
    f = face_in
    for k in range(4):
        nn = int(n / 1.5)
        a, b = _pool_bounds(n, nn)
        vt_infos.append((n, nn, a, b))
        fn = int(f.shape[0] / 1.5)
        f = ((f * nn) // n)[:fn]
        levels_f.append(f)
        Fs.append(fn)
        n = nn
        ns.append(n)
    npads = [_padn(m) for m in ns]
    Fpads = [_padn(m) for m in Fs]

    fidx = [tuple(_pad_rows(levels_f[k][:, j], Fpads[k]) for j in range(3))
            for k in range(5)]

    vs = [_pad_rows(jnp.pad(verts, ((0, 0), (0, 13))), npads[0])]
    for k in range(4):
        n, nn, a, b = vt_infos[k]
        i = jnp.arange(n, dtype=jnp.int32)
        t = (i * nn) // n
        cnt_src = (((t + 1) * n + nn - 1) // nn - (t * n + nn - 1) // nn
                   ).astype(jnp.float32)
        vsrc = _pad_rows(vs[k][:n] * (1.0 / cnt_src)[:, None], npads[k])
        apad = _pad_rows(a, npads[k + 1])
        bpad = _pad_rows(jnp.where(b > a, b, n), npads[k + 1])
        vs.append(sc_gather_comb(vsrc, [apad, bpad], 'sum'))

    coeffs = []
    inv_nfs = []
    geo0 = None
    for k in range(5):
        g3 = sc_gather(vs[k], list(fidx[k]))
        V1 = g3[0][:, :3]
        V2 = g3[1][:, :3]
        V3 = g3[2][:, :3]
        coeff, geo = _face_geom(V1, V2, V3, with_geo=(k == 0))
        if k == 0:
            geo0 = jnp.pad(geo, ((0, 0), (0, 16 - 9)))
        coeffs.append(coeff)
        ones = _pad_rows(jnp.ones((Fs[k], 16), jnp.float32), Fpads[k])
        nfp = sc_scatter_add3(ones, *fidx[k], npads[k])
        inv_nfs.append(nf_inverse(nfp))

    def prep_W(W):
        cin = W.shape[1]
        return W.transpose(1, 0, 2).reshape(cin, K * G)

    def conv_layer(ff, k, W, b, third=True):
        Wp = prep_W(W)
        if third:
            Wp = Wp / 3.0
        msg = conv_matmul(ff, coeffs[k], Wp, b, Fs[k])
        p = sc_scatter_add3(msg, *fidx[k], npads[k])
        return combine_partials(p, inv_nfs[k], relu=True)

    W0 = jnp.pad(params['conv0_W'], ((0, 0), (0, 16 - 9), (0, 0)))
    feats = conv_layer(geo0, 0, W0, params['conv0_b'], third=False)

    skips = []
    for k in range(5):
        bp = params['blocks'][k]
        ff = None
        new = feats
        for W, b in zip(bp['Ws'], bp['bs']):
            ffnew = sc_gather_comb(new, list(fidx[k]), 'sum')
            ff = ffnew if ff is None else jnp.concatenate([ff, ffnew], axis=1)
            new = conv_layer(ff, k, W, b)
            feats = jnp.concatenate([feats, new], axis=1)
        feats = dense(feats, bp['Wout'], bp['bout'], relu=True)
        if k < 4:
            skips.append(feats)
            n, nn, a, b = vt_infos[k]
            apad = _pad_rows(a, npads[k + 1])
            bpad = _pad_rows(b, npads[k + 1])
            feats = sc_gather_comb(feats, [apad, bpad], 'max')

    for k in range(4):
        it = 4 - k
        n, nn, a, b = vt_infos[it - 1]
        vt_map = (jnp.arange(n, dtype=jnp.int32) * nn) // n
        up = sc_gather(feats, [_pad_rows(vt_map, npads[it - 1])])[0]
        feats = jnp.concatenate([skips[it - 1], up], axis=1)
        W, b = params['dec'][k]
        feats = dense(feats, W, b, relu=True)

    predW = jnp.pad(params['pred_W'], ((0, 0), (0, 128 - 13)))
    predb = jnp.pad(params['pred_b'], (0, 128 - 13))
    out = dense(feats, predW, predb, relu=False)
    return out[:ns[0], :13]


# speedup vs baseline: 1.1630x; 1.0295x over previous
import functools
from functools import partial

import jax
import jax.numpy as jnp
from jax import lax
from jax.experimental import pallas as pl
from jax.experimental.pallas import tpu as pltpu
from jax.experimental.pallas import tpu_sc as plsc

K = 16
G = 32

NC, NS = 2, 16
NW = NC * NS
_P = 2048


def _cdiv(a, b):
    return (a + b - 1) // b


def _padn(m):
    return _cdiv(m, _P) * _P


def _pad_rows(x, rows, fill=0):
    return jnp.pad(x, ((0, rows - x.shape[0]),) + ((0, 0),) * (x.ndim - 1),
                   constant_values=fill)


def _pick_bc(rows_pw, cap=128):
    cap = max(8, min(128, cap))
    for bc in range(cap - cap % 8, 7, -8):
        if rows_pw % bc == 0:
            return bc
    return 8


def _sc_mesh():
    return plsc.VectorSubcoreMesh(core_axis_name="c", subcore_axis_name="s",
                                  num_cores=NC, num_subcores=NS)


def sc_gather(src, idx_list):
    S, C = src.shape
    nidx = len(idx_list)
    Fpad = idx_list[0].shape[0]
    rows_pw = Fpad // NW
    BC = _pick_bc(rows_pw, 100000 // (2 * nidx * C))
    nchunks = rows_pw // BC
    scratch = ([pltpu.VMEM((rows_pw,), jnp.int32)] * nidx
               + [pltpu.VMEM((BC, C), jnp.float32)] * (2 * nidx)
               + [pltpu.SemaphoreType.DMA, pltpu.SemaphoreType.DMA])

    def body(*args):
        src_hbm = args[0]
        idx_hbm = args[1:1 + nidx]
        out_hbm = args[1 + nidx]
        idx_v = args[2 + nidx:2 + 2 * nidx]
        bufs = [args[2 + 2 * nidx:2 + 3 * nidx],
                args[2 + 3 * nidx:2 + 4 * nidx]]
        gsem, osem = args[-2], args[-1]
        wid = lax.axis_index("s") * NC + lax.axis_index("c")
        base = wid * rows_pw
        for j in range(nidx):
            pltpu.sync_copy(idx_hbm[j].at[pl.ds(base, rows_pw)], idx_v[j])

        def fire_gather(jc):
            off = jc * BC
            sl = bufs[jc % 2]
            return [pltpu.async_copy(
                src_hbm.at[idx_v[j].at[pl.ds(off, BC)]], sl[j], gsem)
                for j in range(nidx)]

        def fire_out(jc):
            off = jc * BC
            sl = bufs[jc % 2]
            return [pltpu.async_copy(
                sl[j], out_hbm.at[j, pl.ds(base + off, BC)], osem)
                for j in range(nidx)]

        gd = {0: fire_gather(0)}
        od = {}
        for jc in range(nchunks):
            if jc + 1 < nchunks:
                if jc >= 1:
                    for d in od.pop(jc - 1):
                        d.wait()
                gd[jc + 1] = fire_gather(jc + 1)
            for d in gd.pop(jc):
                d.wait()
            od[jc] = fire_out(jc)
        for descs in od.values():
            for d in descs:
                d.wait()

    return pl.kernel(
        body,
        out_type=jax.ShapeDtypeStruct((nidx, Fpad, C), jnp.float32),
        mesh=_sc_mesh(),
        scratch_types=scratch,
        compiler_params=pltpu.CompilerParams(use_tc_tiling_on_sc=False),
        name=f"scg{nidx}_c{C}_f{Fpad}",
    )(src, *idx_list)


def sc_gather_comb(src, idx_list, op):
    S, C = src.shape
    nidx = len(idx_list)
    Fpad = idx_list[0].shape[0]
    rows_pw = Fpad // NW
    BC = _pick_bc(rows_pw, 100000 // (nidx * C))
    nchunks = rows_pw // BC
    scratch = ([pltpu.VMEM((rows_pw,), jnp.int32)] * nidx
               + [pltpu.VMEM((BC, C), jnp.float32)] * nidx
               + [pltpu.SemaphoreType.DMA])

    def body(*args):
        src_hbm = args[0]
        idx_hbm = args[1:1 + nidx]
        out_hbm = args[1 + nidx]
        idx_v = args[2 + nidx:2 + 2 * nidx]
        bufs = args[2 + 2 * nidx:2 + 3 * nidx]
        sem = args[-1]
        wid = lax.axis_index("s") * NC + lax.axis_index("c")
        base = wid * rows_pw
        for j in range(nidx):
            pltpu.sync_copy(idx_hbm[j].at[pl.ds(base, rows_pw)], idx_v[j])

        def chunk(jc, carry):
            off = jc * BC
            cps = [pltpu.async_copy(src_hbm.at[idx_v[j].at[pl.ds(off, BC)]],
                                    bufs[j], sem) for j in range(nidx)]
            for cp in cps:
                cp.wait()

            def row(r, c2):
                for c0 in range(C // 16):
                    sl = pl.ds(c0 * 16, 16)
                    if op == 'sum':
                        acc = bufs[0][r, sl] + bufs[1][r, sl]
                        if nidx == 3:
                            acc = acc + bufs[2][r, sl]
                        bufs[0][r, sl] = acc
                    else:
                        bufs[0][r, sl] = jnp.maximum(bufs[0][r, sl],
                                                     bufs[1][r, sl])
                return c2
            lax.fori_loop(0, BC, row, 0)
            pltpu.sync_copy(bufs[0], out_hbm.at[pl.ds(base + off, BC)])
            return carry
        lax.fori_loop(0, nchunks, chunk, 0)

    return pl.kernel(
        body,
        out_type=jax.ShapeDtypeStruct((Fpad, C), jnp.float32),
        mesh=_sc_mesh(),
        scratch_types=scratch,
        compiler_params=pltpu.CompilerParams(use_tc_tiling_on_sc=False),
        name=f"scgc_{op}{nidx}_c{C}_f{Fpad}",
    )(src, *idx_list)


def _red_body(op, x_ref, out_ref):
    x = x_ref[...]
    acc = x[0]
    for j in range(1, x.shape[0]):
        acc = acc + x[j] if op == 'sum' else jnp.maximum(acc, x[j])
    out_ref[...] = acc


def comb_op(x, op, bm=2048):
    nidx, npad, C = x.shape
    return pl.pallas_call(
        partial(_red_body, op),
        grid=(_cdiv(npad, bm),),
        in_specs=[pl.BlockSpec((nidx, bm, C), lambda i: (0, i, 0))],
        out_specs=pl.BlockSpec((bm, C), lambda i: (i, 0)),
        out_shape=jax.ShapeDtypeStruct((npad, C), jnp.float32),
    )(x)


def sc_scatter_add3(msg, i0, i1, i2, npad):
    Fpad, GC = msg.shape
    rows_pw = Fpad // NW
    BC = _pick_bc(rows_pw)
    nchunks = rows_pw // BC
    i0r = i0.reshape(Fpad // BC, BC)
    i1r = i1.reshape(Fpad // BC, BC)
    i2r = i2.reshape(Fpad // BC, BC)
    BCZ = 128
    zrows = npad // NS
    nz = zrows // BCZ
    scratch = ([pltpu.VMEM((nchunks, BC), jnp.int32)] * 3
               + [pltpu.VMEM((BC, GC), jnp.float32)] * 2
               + [pltpu.VMEM((BCZ, GC), jnp.float32),
                  pltpu.VMEM_SHARED((npad, GC), jnp.float32),
                  pltpu.SemaphoreType.DMA, pltpu.SemaphoreType.DMA])

    def body(msg_hbm, i0h, i1h, i2h, out_hbm, iv0, iv1, iv2, mb0, mb1, zbuf,
             shared, msem, ssem):
        cid = lax.axis_index("c")
        sid = lax.axis_index("s")
        wid = sid * NC + cid
        mbufs = (mb0, mb1)

        def zrow(r, c2):
            for c0 in range(GC // 16):
                zbuf[r, pl.ds(c0 * 16, 16)] = jnp.zeros((16,), jnp.float32)
            return c2
        lax.fori_loop(0, BCZ, zrow, 0)
        zd = [pltpu.async_copy(
            zbuf, shared.at[pl.ds(sid * zrows + j * BCZ, BCZ)], msem)
            for j in range(nz)]
        for d in zd:
            d.wait()
        plsc.subcore_barrier()

        chunk0 = wid * nchunks
        pltpu.sync_copy(i0h.at[pl.ds(chunk0, nchunks)], iv0)
        pltpu.sync_copy(i1h.at[pl.ds(chunk0, nchunks)], iv1)
        pltpu.sync_copy(i2h.at[pl.ds(chunk0, nchunks)], iv2)

        def fire_msg(jc):
            return pltpu.async_copy(
                msg_hbm.at[pl.ds((chunk0 + jc) * BC, BC)], mbufs[jc % 2], msem)

        def fire_scat(jc):
            return [pltpu.async_copy(mbufs[jc % 2], shared.at[iv.at[jc]],
                                     ssem, add=True)
                    for iv in (iv0, iv1, iv2)]

        md = {0: fire_msg(0)}
        sd = {}
        for jc in range(nchunks):
            if jc + 1 < nchunks:
                if jc >= 1:
                    for d in sd.pop(jc - 1):
                        d.wait()
                md[jc + 1] = fire_msg(jc + 1)
            md.pop(jc).wait()
            sd[jc] = fire_scat(jc)
        for descs in sd.values():
            for d in descs:
                d.wait()
        plsc.subcore_barrier()

        od = [pltpu.async_copy(
            shared.at[pl.ds(sid * zrows + j * BCZ, BCZ)],
            out_hbm.at[cid, pl.ds(sid * zrows + j * BCZ, BCZ)], msem)
            for j in range(nz)]
        for d in od:
            d.wait()

    return pl.kernel(
        body,
        out_type=jax.ShapeDtypeStruct((2, npad, GC), jnp.float32),
        mesh=_sc_mesh(),
        scratch_types=scratch,
        compiler_params=pltpu.CompilerParams(use_tc_tiling_on_sc=False),
        name=f"scs_g{GC}_f{Fpad}_n{npad}",
    )(msg, i0r, i1r, i2r)


def _comb_body(relu, p0_ref, p1_ref, inv_ref, out_ref):
    acc = (p0_ref[0] + p1_ref[0]) * inv_ref[...]
    if relu:
        acc = jnp.maximum(acc, 0.0)
    out_ref[...] = acc


def combine_partials(p, inv_nf, relu, bm=1024):
    _, npad, GC = p.shape
    return pl.pallas_call(
        partial(_comb_body, relu),
        grid=(_cdiv(npad, bm),),
        in_specs=[
            pl.BlockSpec((1, bm, GC), lambda i: (0, i, 0)),
            pl.BlockSpec((1, bm, GC), lambda i: (1, i, 0)),
            pl.BlockSpec((bm, GC), lambda i: (i, 0)),
        ],
        out_specs=pl.BlockSpec((bm, GC), lambda i: (i, 0)),
        out_shape=jax.ShapeDtypeStruct((npad, GC), jnp.float32),
    )(p, p, inv_nf)


def _nf_body(p0_ref, p1_ref, out_ref):
    s = p0_ref[0, :, 0:1] + p1_ref[0, :, 0:1]
    s = jnp.maximum(s, 1.0)
    out_ref[...] = jnp.broadcast_to(1.0 / s, out_ref.shape)


def nf_inverse(p, bm=1024):
    _, npad, GC = p.shape
    return pl.pallas_call(
        _nf_body,
        grid=(_cdiv(npad, bm),),
        in_specs=[
            pl.BlockSpec((1, bm, GC), lambda i: (0, i, 0)),
            pl.BlockSpec((1, bm, GC), lambda i: (1, i, 0)),
        ],
        out_specs=pl.BlockSpec((bm, G), lambda i: (i, 0)),
        out_shape=jax.ShapeDtypeStruct((npad, G), jnp.float32),
    )(p, p)


def _conv_body(nrows, bf, ff_ref, coeff_ref, wp_ref, b_ref, out_ref):
    ff = ff_ref[...]
    if ff.ndim == 3:
        ff = ff[0] + ff[1] + ff[2]
    t = jnp.dot(ff, wp_ref[...], preferred_element_type=jnp.float32)
    coeff = coeff_ref[...]
    acc = b_ref[0, :][None, :] + coeff[:, 0:1] * t[:, 0:G]
    for k in range(1, K):
        acc = acc + coeff[:, k:k + 1] * t[:, k * G:(k + 1) * G]
    rid = pl.program_id(0) * bf + jax.lax.broadcasted_iota(jnp.int32, acc.shape, 0)
    out_ref[...] = jnp.where(rid < nrows, acc, 0.0)


def conv_matmul(ff, coeff, Wp, b, nrows, bf=1024):
    if ff.ndim == 3:
        _, F, cin = ff.shape
        ff_spec = pl.BlockSpec((3, bf, cin), lambda i: (0, i, 0))
    else:
        F, cin = ff.shape
        ff_spec = pl.BlockSpec((bf, cin), lambda i: (i, 0))
    grid = (_cdiv(F, bf),)
    return pl.pallas_call(
        partial(_conv_body, nrows, bf),
        grid=grid,
        in_specs=[
            ff_spec,
            pl.BlockSpec((bf, K), lambda i: (i, 0)),
            pl.BlockSpec((cin, K * G), lambda i: (0, 0)),
            pl.BlockSpec((1, G), lambda i: (0, 0)),
        ],
        out_specs=pl.BlockSpec((bf, G), lambda i: (i, 0)),
        out_shape=jax.ShapeDtypeStruct((F, G), jnp.float32),
    )(ff, coeff, Wp, b.reshape(1, G))


def _mm_body(relu, x_ref, w_ref, b_ref, out_ref):
    acc = jnp.dot(x_ref[...], w_ref[...], preferred_element_type=jnp.float32)
    acc = acc + b_ref[0, :][None, :]
    if relu:
        acc = jnp.maximum(acc, 0.0)
    out_ref[...] = acc


def dense(x, W, b, relu=True, bm=1024):
    M, Kd = x.shape
    N = W.shape[1]
    return pl.pallas_call(
        partial(_mm_body, relu),
        grid=(_cdiv(M, bm),),
        in_specs=[
            pl.BlockSpec((bm, Kd), lambda i: (i, 0)),
            pl.BlockSpec((Kd, N), lambda i: (0, 0)),
            pl.BlockSpec((1, N), lambda i: (0, 0)),
        ],
        out_specs=pl.BlockSpec((bm, N), lambda i: (i, 0)),
        out_shape=jax.ShapeDtypeStruct((M, N), jnp.float32),
    )(x, W, b.reshape(1, N))


def _pool_bounds(n, nn):
    t = jnp.arange(nn, dtype=jnp.int32)
    a = (t * n + nn - 1) // nn
    b = ((t + 1) * n + nn - 1) // nn - 1
    return a, b


def _sph_harm_coeff(n):
    x = n[:, 0]; y = n[:, 1]; z = n[:, 2]
    x2 = x * x; y2 = y * y; z2 = z * z
    c = [
        0.28209479177387814 * jnp.ones_like(x),
        0.4886025119029199 * y,
        0.4886025119029199 * z,
        0.4886025119029199 * x,
        1.0925484305920792 * x * y,
        1.0925484305920792 * y * z,
        0.31539156525252005 * (3.0 * z2 - 1.0),
        1.0925484305920792 * x * z,
        0.5462742152960396 * (x2 - y2),
        0.5900435899266435 * y * (3.0 * x2 - y2),
        2.890611442640554 * x * y * z,
        0.4570457994644658 * y * (5.0 * z2 - 1.0),
        0.3731763325901154 * z * (5.0 * z2 - 3.0),
        0.4570457994644658 * x * (5.0 * z2 - 1.0),
        1.445305721320277 * z * (x2 - y2),
        0.5900435899266435 * x * (x2 - 3.0 * y2),
    ]
    return jnp.stack(c, axis=1)


def _snorm(v):
    return jnp.sqrt(jnp.sum(v * v, axis=-1, keepdims=True) + 1e-12)


def _face_geom(V1, V2, V3, with_geo):
    nrm = jnp.cross(V2 - V1, V3 - V1)
    nrm = nrm / _snorm(nrm)
    coeff = _sph_harm_coeff(nrm)
    if not with_geo:
        return coeff, None
    D12 = V2 - V1; D23 = V3 - V2; D31 = V1 - V3
    L12 = _snorm(D12); L23 = _snorm(D23); L31 = _snorm(D31)
    eps = 1e-8
    T1 = jnp.sum(D12 * -D31, axis=-1, keepdims=True) / (L12 * L31 + eps)
    T2 = jnp.sum(-D12 * D23, axis=-1, keepdims=True) / (L12 * L23 + eps)
    T3 = jnp.sum(-D23 * D31, axis=-1, keepdims=True) / (L23 * L31 + eps)
    geo = jnp.concatenate([L12, L23, L31, T1, T2, T3, nrm], axis=-1)
    return coeff, geo


def kernel(vertex_in, face_in, nv_in, mf_in, params):
    verts = vertex_in[:, :3]

    ns = [verts.shape[0]]
    Fs = [face_in.shape[0]]
    levels_f = [face_in]
    vt_infos = []
    n = ns[0]
    f = face_in
    for k in range(4):
        nn = int(n / 1.5)
        a, b = _pool_bounds(n, nn)
        vt_infos.append((n, nn, a, b))
        fn = int(f.shape[0] / 1.5)
        f = ((f * nn) // n)[:fn]
        levels_f.append(f)
        Fs.append(fn)
        n = nn
        ns.append(n)
    npads = [_padn(m) for m in ns]
    Fpads = [_padn(m) for m in Fs]

    fidx = [tuple(_pad_rows(levels_f[k][:, j], Fpads[k]) for j in range(3))
            for k in range(5)]

    vs = [_pad_rows(jnp.pad(verts, ((0, 0), (0, 13))), npads[0])]
    for k in range(4):
        n, nn, a, b = vt_infos[k]
        i = jnp.arange(n, dtype=jnp.int32)
        t = (i * nn) // n
        cnt_src = (((t + 1) * n + nn - 1) // nn - (t * n + nn - 1) // nn
                   ).astype(jnp.float32)
        vsrc = _pad_rows(vs[k][:n] * (1.0 / cnt_src)[:, None], npads[k])
        apad = _pad_rows(a, npads[k + 1])
        bpad = _pad_rows(jnp.where(b > a, b, n), npads[k + 1])
        vs.append(sc_gather_comb(vsrc, [apad, bpad], 'sum'))

    coeffs = []
    inv_nfs = []
    geo0 = None
    for k in range(5):
        g3 = sc_gather(vs[k], list(fidx[k]))
        V1 = g3[0][:, :3]
        V2 = g3[1][:, :3]
        V3 = g3[2][:, :3]
        coeff, geo = _face_geom(V1, V2, V3, with_geo=(k == 0))
        if k == 0:
            geo0 = jnp.pad(geo, ((0, 0), (0, 16 - 9)))
        coeffs.append(coeff)
        ones = _pad_rows(jnp.ones((Fs[k], 16), jnp.float32), Fpads[k])
        nfp = sc_scatter_add3(ones, *fidx[k], npads[k])
        inv_nfs.append(nf_inverse(nfp))

    def prep_W(W):
        cin = W.shape[1]
        return W.transpose(1, 0, 2).reshape(cin, K * G)

    def conv_layer(ff, k, W, b, third=True):
        Wp = prep_W(W)
        if third:
            Wp = Wp / 3.0
        msg = conv_matmul(ff, coeffs[k], Wp, b, Fs[k])
        p = sc_scatter_add3(msg, *fidx[k], npads[k])
        return combine_partials(p, inv_nfs[k], relu=True)

    W0 = jnp.pad(params['conv0_W'], ((0, 0), (0, 16 - 9), (0, 0)))
    feats = conv_layer(geo0, 0, W0, params['conv0_b'], third=False)

    skips = []
    for k in range(5):
        bp = params['blocks'][k]
        ff = None
        new = feats
        for W, b in zip(bp['Ws'], bp['bs']):
            ffnew = sc_gather_comb(new, list(fidx[k]), 'sum')
            ff = ffnew if ff is None else jnp.concatenate([ff, ffnew], axis=1)
            new = conv_layer(ff, k, W, b)
            feats = jnp.concatenate([feats, new], axis=1)
        feats = dense(feats, bp['Wout'], bp['bout'], relu=True)
        if k < 4:
            skips.append(feats)
            n, nn, a, b = vt_infos[k]
            apad = _pad_rows(a, npads[k + 1])
            bpad = _pad_rows(b, npads[k + 1])
            feats = sc_gather_comb(feats, [apad, bpad], 'max')

    for k in range(4):
        it = 4 - k
        n, nn, a, b = vt_infos[it - 1]
        vt_map = (jnp.arange(n, dtype=jnp.int32) * nn) // n
        up = sc_gather(feats, [_pad_rows(vt_map, npads[it - 1])])[0]
        feats = jnp.concatenate([skips[it - 1], up], axis=1)
        W, b = params['dec'][k]
        feats = dense(feats, W, b, relu=True)

    predW = jnp.pad(params['pred_W'], ((0, 0), (0, 128 - 13)))
    predb = jnp.pad(params['pred_b'], (0, 128 - 13))
    out = dense(feats, predW, predb, relu=False)
    return out[:ns[0], :13]


# speedup vs baseline: 1.1841x; 1.0182x over previous
import functools
from functools import partial

import jax
import jax.numpy as jnp
from jax import lax
from jax.experimental import pallas as pl
from jax.experimental.pallas import tpu as pltpu
from jax.experimental.pallas import tpu_sc as plsc

K = 16
G = 32

NC, NS = 2, 16
NW = NC * NS
_P = 2048


def _cdiv(a, b):
    return (a + b - 1) // b


def _padn(m):
    return _cdiv(m, _P) * _P


def _pad_rows(x, rows, fill=0):
    return jnp.pad(x, ((0, rows - x.shape[0]),) + ((0, 0),) * (x.ndim - 1),
                   constant_values=fill)


def _pick_bc(rows_pw, cap=128):
    cap = max(8, min(128, cap))
    for bc in range(cap - cap % 8, 7, -8):
        if rows_pw % bc == 0:
            return bc
    return 8


def _sc_mesh():
    return plsc.VectorSubcoreMesh(core_axis_name="c", subcore_axis_name="s",
                                  num_cores=NC, num_subcores=NS)


def sc_gather(src, idx_list):
    S, C = src.shape
    nidx = len(idx_list)
    Fpad = idx_list[0].shape[0]
    rows_pw = Fpad // NW
    BC = _pick_bc(rows_pw, 100000 // (2 * nidx * C))
    nchunks = rows_pw // BC
    scratch = ([pltpu.VMEM((rows_pw,), jnp.int32)] * nidx
               + [pltpu.VMEM((BC, C), jnp.float32)] * (2 * nidx)
               + [pltpu.SemaphoreType.DMA, pltpu.SemaphoreType.DMA])

    def body(*args):
        src_hbm = args[0]
        idx_hbm = args[1:1 + nidx]
        out_hbm = args[1 + nidx]
        idx_v = args[2 + nidx:2 + 2 * nidx]
        bufs = [args[2 + 2 * nidx:2 + 3 * nidx],
                args[2 + 3 * nidx:2 + 4 * nidx]]
        gsem, osem = args[-2], args[-1]
        wid = lax.axis_index("s") * NC + lax.axis_index("c")
        base = wid * rows_pw
        for j in range(nidx):
            pltpu.sync_copy(idx_hbm[j].at[pl.ds(base, rows_pw)], idx_v[j])

        def fire_gather(jc):
            off = jc * BC
            sl = bufs[jc % 2]
            return [pltpu.async_copy(
                src_hbm.at[idx_v[j].at[pl.ds(off, BC)]], sl[j], gsem)
                for j in range(nidx)]

        def fire_out(jc):
            off = jc * BC
            sl = bufs[jc % 2]
            return [pltpu.async_copy(
                sl[j], out_hbm.at[j, pl.ds(base + off, BC)], osem)
                for j in range(nidx)]

        gd = {0: fire_gather(0)}
        od = {}
        for jc in range(nchunks):
            if jc + 1 < nchunks:
                if jc >= 1:
                    for d in od.pop(jc - 1):
                        d.wait()
                gd[jc + 1] = fire_gather(jc + 1)
            for d in gd.pop(jc):
                d.wait()
            od[jc] = fire_out(jc)
        for descs in od.values():
            for d in descs:
                d.wait()

    return pl.kernel(
        body,
        out_type=jax.ShapeDtypeStruct((nidx, Fpad, C), jnp.float32),
        mesh=_sc_mesh(),
        scratch_types=scratch,
        compiler_params=pltpu.CompilerParams(use_tc_tiling_on_sc=False),
        name=f"scg{nidx}_c{C}_f{Fpad}",
    )(src, *idx_list)


def sc_gather_comb(src, idx_list, op):
    S, C = src.shape
    nidx = len(idx_list)
    Fpad = idx_list[0].shape[0]
    rows_pw = Fpad // NW
    BC = _pick_bc(rows_pw, 100000 // (nidx * C))
    nchunks = rows_pw // BC
    scratch = ([pltpu.VMEM((rows_pw,), jnp.int32)] * nidx
               + [pltpu.VMEM((BC, C), jnp.float32)] * nidx
               + [pltpu.SemaphoreType.DMA])

    def body(*args):
        src_hbm = args[0]
        idx_hbm = args[1:1 + nidx]
        out_hbm = args[1 + nidx]
        idx_v = args[2 + nidx:2 + 2 * nidx]
        bufs = args[2 + 2 * nidx:2 + 3 * nidx]
        sem = args[-1]
        wid = lax.axis_index("s") * NC + lax.axis_index("c")
        base = wid * rows_pw
        for j in range(nidx):
            pltpu.sync_copy(idx_hbm[j].at[pl.ds(base, rows_pw)], idx_v[j])

        def chunk(jc, carry):
            off = jc * BC
            cps = [pltpu.async_copy(src_hbm.at[idx_v[j].at[pl.ds(off, BC)]],
                                    bufs[j], sem) for j in range(nidx)]
            for cp in cps:
                cp.wait()

            def row(r, c2):
                for c0 in range(C // 16):
                    sl = pl.ds(c0 * 16, 16)
                    if op == 'sum':
                        acc = bufs[0][r, sl] + bufs[1][r, sl]
                        if nidx == 3:
                            acc = acc + bufs[2][r, sl]
                        bufs[0][r, sl] = acc
                    else:
                        bufs[0][r, sl] = jnp.maximum(bufs[0][r, sl],
                                                     bufs[1][r, sl])
                return c2
            lax.fori_loop(0, BC, row, 0)
            pltpu.sync_copy(bufs[0], out_hbm.at[pl.ds(base + off, BC)])
            return carry
        lax.fori_loop(0, nchunks, chunk, 0)

    return pl.kernel(
        body,
        out_type=jax.ShapeDtypeStruct((Fpad, C), jnp.float32),
        mesh=_sc_mesh(),
        scratch_types=scratch,
        compiler_params=pltpu.CompilerParams(use_tc_tiling_on_sc=False),
        name=f"scgc_{op}{nidx}_c{C}_f{Fpad}",
    )(src, *idx_list)


def _red_body(op, x_ref, out_ref):
    x = x_ref[...]
    acc = x[0]
    for j in range(1, x.shape[0]):
        acc = acc + x[j] if op == 'sum' else jnp.maximum(acc, x[j])
    out_ref[...] = acc


def comb_op(x, op, bm=2048):
    nidx, npad, C = x.shape
    return pl.pallas_call(
        partial(_red_body, op),
        grid=(_cdiv(npad, bm),),
        in_specs=[pl.BlockSpec((nidx, bm, C), lambda i: (0, i, 0))],
        out_specs=pl.BlockSpec((bm, C), lambda i: (i, 0)),
        out_shape=jax.ShapeDtypeStruct((npad, C), jnp.float32),
    )(x)


def sc_scatter_add3(msg, i0, i1, i2, npad):
    Fpad, GC = msg.shape
    rows_pw = Fpad // NW
    BC = _pick_bc(rows_pw)
    nchunks = rows_pw // BC
    i0r = i0.reshape(Fpad // BC, BC)
    i1r = i1.reshape(Fpad // BC, BC)
    i2r = i2.reshape(Fpad // BC, BC)
    BCZ = 128
    zrows = npad // NS
    nz = zrows // BCZ
    scratch = ([pltpu.VMEM((nchunks, BC), jnp.int32)] * 3
               + [pltpu.VMEM((BC, GC), jnp.float32)] * 2
               + [pltpu.VMEM((BCZ, GC), jnp.float32),
                  pltpu.VMEM_SHARED((npad, GC), jnp.float32),
                  pltpu.SemaphoreType.DMA, pltpu.SemaphoreType.DMA])

    def body(msg_hbm, i0h, i1h, i2h, out_hbm, iv0, iv1, iv2, mb0, mb1, zbuf,
             shared, msem, ssem):
        cid = lax.axis_index("c")
        sid = lax.axis_index("s")
        wid = sid * NC + cid
        mbufs = (mb0, mb1)

        def zrow(r, c2):
            for c0 in range(GC // 16):
                zbuf[r, pl.ds(c0 * 16, 16)] = jnp.zeros((16,), jnp.float32)
            return c2
        lax.fori_loop(0, BCZ, zrow, 0)
        zd = [pltpu.async_copy(
            zbuf, shared.at[pl.ds(sid * zrows + j * BCZ, BCZ)], msem)
            for j in range(nz)]
        for d in zd:
            d.wait()
        plsc.subcore_barrier()

        chunk0 = wid * nchunks
        pltpu.sync_copy(i0h.at[pl.ds(chunk0, nchunks)], iv0)
        pltpu.sync_copy(i1h.at[pl.ds(chunk0, nchunks)], iv1)
        pltpu.sync_copy(i2h.at[pl.ds(chunk0, nchunks)], iv2)

        def fire_msg(jc):
            return pltpu.async_copy(
                msg_hbm.at[pl.ds((chunk0 + jc) * BC, BC)], mbufs[jc % 2], msem)

        def fire_scat(jc):
            return [pltpu.async_copy(mbufs[jc % 2], shared.at[iv.at[jc]],
                                     ssem, add=True)
                    for iv in (iv0, iv1, iv2)]

        md = {0: fire_msg(0)}
        sd = {}
        for jc in range(nchunks):
            if jc + 1 < nchunks:
                if jc >= 1:
                    for d in sd.pop(jc - 1):
                        d.wait()
                md[jc + 1] = fire_msg(jc + 1)
            md.pop(jc).wait()
            sd[jc] = fire_scat(jc)
        for descs in sd.values():
            for d in descs:
                d.wait()
        plsc.subcore_barrier()

        od = [pltpu.async_copy(
            shared.at[pl.ds(sid * zrows + j * BCZ, BCZ)],
            out_hbm.at[cid, pl.ds(sid * zrows + j * BCZ, BCZ)], msem)
            for j in range(nz)]
        for d in od:
            d.wait()

    return pl.kernel(
        body,
        out_type=jax.ShapeDtypeStruct((2, npad, GC), jnp.float32),
        mesh=_sc_mesh(),
        scratch_types=scratch,
        compiler_params=pltpu.CompilerParams(use_tc_tiling_on_sc=False),
        name=f"scs_g{GC}_f{Fpad}_n{npad}",
    )(msg, i0r, i1r, i2r)


def _comb_body(relu, p0_ref, p1_ref, inv_ref, out_ref):
    acc = (p0_ref[0] + p1_ref[0]) * inv_ref[...]
    if relu:
        acc = jnp.maximum(acc, 0.0)
    out_ref[...] = acc


def combine_partials(p, inv_nf, relu, bm=2048):
    _, npad, GC = p.shape
    return pl.pallas_call(
        partial(_comb_body, relu),
        grid=(_cdiv(npad, bm),),
        in_specs=[
            pl.BlockSpec((1, bm, GC), lambda i: (0, i, 0)),
            pl.BlockSpec((1, bm, GC), lambda i: (1, i, 0)),
            pl.BlockSpec((bm, GC), lambda i: (i, 0)),
        ],
        out_specs=pl.BlockSpec((bm, GC), lambda i: (i, 0)),
        out_shape=jax.ShapeDtypeStruct((npad, GC), jnp.float32),
    )(p, p, inv_nf)


def _nf_body(p0_ref, p1_ref, out_ref):
    s = p0_ref[0, :, 0:1] + p1_ref[0, :, 0:1]
    s = jnp.maximum(s, 1.0)
    out_ref[...] = jnp.broadcast_to(1.0 / s, out_ref.shape)


def nf_inverse(p, bm=1024):
    _, npad, GC = p.shape
    return pl.pallas_call(
        _nf_body,
        grid=(_cdiv(npad, bm),),
        in_specs=[
            pl.BlockSpec((1, bm, GC), lambda i: (0, i, 0)),
            pl.BlockSpec((1, bm, GC), lambda i: (1, i, 0)),
        ],
        out_specs=pl.BlockSpec((bm, G), lambda i: (i, 0)),
        out_shape=jax.ShapeDtypeStruct((npad, G), jnp.float32),
    )(p, p)


def _conv_body(nrows, bf, ff_ref, coeff_ref, wp_ref, b_ref, out_ref):
    ff = ff_ref[...]
    if ff.ndim == 3:
        ff = ff[0] + ff[1] + ff[2]
    t = jnp.dot(ff, wp_ref[...], preferred_element_type=jnp.float32)
    coeff = coeff_ref[...]
    acc = b_ref[0, :][None, :] + coeff[:, 0:1] * t[:, 0:G]
    for k in range(1, K):
        acc = acc + coeff[:, k:k + 1] * t[:, k * G:(k + 1) * G]
    rid = pl.program_id(0) * bf + jax.lax.broadcasted_iota(jnp.int32, acc.shape, 0)
    out_ref[...] = jnp.where(rid < nrows, acc, 0.0)


def conv_matmul(ff, coeff, Wp, b, nrows, bf=2048):
    if ff.ndim == 3:
        _, F, cin = ff.shape
        ff_spec = pl.BlockSpec((3, bf, cin), lambda i: (0, i, 0))
    else:
        F, cin = ff.shape
        ff_spec = pl.BlockSpec((bf, cin), lambda i: (i, 0))
    grid = (_cdiv(F, bf),)
    return pl.pallas_call(
        partial(_conv_body, nrows, bf),
        grid=grid,
        in_specs=[
            ff_spec,
            pl.BlockSpec((bf, K), lambda i: (i, 0)),
            pl.BlockSpec((cin, K * G), lambda i: (0, 0)),
            pl.BlockSpec((1, G), lambda i: (0, 0)),
        ],
        out_specs=pl.BlockSpec((bf, G), lambda i: (i, 0)),
        out_shape=jax.ShapeDtypeStruct((F, G), jnp.float32),
    )(ff, coeff, Wp, b.reshape(1, G))


def _mm_body(relu, x_ref, w_ref, b_ref, out_ref):
    acc = jnp.dot(x_ref[...], w_ref[...], preferred_element_type=jnp.float32)
    acc = acc + b_ref[0, :][None, :]
    if relu:
        acc = jnp.maximum(acc, 0.0)
    out_ref[...] = acc


def dense(x, W, b, relu=True, bm=2048):
    M, Kd = x.shape
    N = W.shape[1]
    return pl.pallas_call(
        partial(_mm_body, relu),
        grid=(_cdiv(M, bm),),
        in_specs=[
            pl.BlockSpec((bm, Kd), lambda i: (i, 0)),
            pl.BlockSpec((Kd, N), lambda i: (0, 0)),
            pl.BlockSpec((1, N), lambda i: (0, 0)),
        ],
        out_specs=pl.BlockSpec((bm, N), lambda i: (i, 0)),
        out_shape=jax.ShapeDtypeStruct((M, N), jnp.float32),
    )(x, W, b.reshape(1, N))


def _pool_bounds(n, nn):
    t = jnp.arange(nn, dtype=jnp.int32)
    a = (t * n + nn - 1) // nn
    b = ((t + 1) * n + nn - 1) // nn - 1
    return a, b


def _sph_harm_coeff(n):
    x = n[:, 0]; y = n[:, 1]; z = n[:, 2]
    x2 = x * x; y2 = y * y; z2 = z * z
    c = [
        0.28209479177387814 * jnp.ones_like(x),
        0.4886025119029199 * y,
        0.4886025119029199 * z,
        0.4886025119029199 * x,
        1.0925484305920792 * x * y,
        1.0925484305920792 * y * z,
        0.31539156525252005 * (3.0 * z2 - 1.0),
        1.0925484305920792 * x * z,
        0.5462742152960396 * (x2 - y2),
        0.5900435899266435 * y * (3.0 * x2 - y2),
        2.890611442640554 * x * y * z,
        0.4570457994644658 * y * (5.0 * z2 - 1.0),
        0.3731763325901154 * z * (5.0 * z2 - 3.0),
        0.4570457994644658 * x * (5.0 * z2 - 1.0),
        1.445305721320277 * z * (x2 - y2),
        0.5900435899266435 * x * (x2 - 3.0 * y2),
    ]
    return jnp.stack(c, axis=1)


def _snorm(v):
    return jnp.sqrt(jnp.sum(v * v, axis=-1, keepdims=True) + 1e-12)


def _face_geom(V1, V2, V3, with_geo):
    nrm = jnp.cross(V2 - V1, V3 - V1)
    nrm = nrm / _snorm(nrm)
    coeff = _sph_harm_coeff(nrm)
    if not with_geo:
        return coeff, None
    D12 = V2 - V1; D23 = V3 - V2; D31 = V1 - V3
    L12 = _snorm(D12); L23 = _snorm(D23); L31 = _snorm(D31)
    eps = 1e-8
    T1 = jnp.sum(D12 * -D31, axis=-1, keepdims=True) / (L12 * L31 + eps)
    T2 = jnp.sum(-D12 * D23, axis=-1, keepdims=True) / (L12 * L23 + eps)
    T3 = jnp.sum(-D23 * D31, axis=-1, keepdims=True) / (L23 * L31 + eps)
    geo = jnp.concatenate([L12, L23, L31, T1, T2, T3, nrm], axis=-1)
    return coeff, geo


def kernel(vertex_in, face_in, nv_in, mf_in, params):
    verts = vertex_in[:, :3]

    ns = [verts.shape[0]]
    Fs = [face_in.shape[0]]
    levels_f = [face_in]
    vt_infos = []
    n = ns[0]
    f = face_in
    for k in range(4):
        nn = int(n / 1.5)
        a, b = _pool_bounds(n, nn)
        vt_infos.append((n, nn, a, b))
        fn = int(f.shape[0] / 1.5)
        f = ((f * nn) // n)[:fn]
        levels_f.append(f)
        Fs.append(fn)
        n = nn
        ns.append(n)
    npads = [_padn(m) for m in ns]
    Fpads = [_padn(m) for m in Fs]

    fidx = [tuple(_pad_rows(levels_f[k][:, j], Fpads[k]) for j in range(3))
            for k in range(5)]

    vs = [_pad_rows(jnp.pad(verts, ((0, 0), (0, 13))), npads[0])]
    for k in range(4):
        n, nn, a, b = vt_infos[k]
        i = jnp.arange(n, dtype=jnp.int32)
        t = (i * nn) // n
        cnt_src = (((t + 1) * n + nn - 1) // nn - (t * n + nn - 1) // nn
                   ).astype(jnp.float32)
        vsrc = _pad_rows(vs[k][:n] * (1.0 / cnt_src)[:, None], npads[k])
        apad = _pad_rows(a, npads[k + 1])
        bpad = _pad_rows(jnp.where(b > a, b, n), npads[k + 1])
        vs.append(sc_gather_comb(vsrc, [apad, bpad], 'sum'))

    coeffs = []
    inv_nfs = []
    geo0 = None
    for k in range(5):
        g3 = sc_gather(vs[k], list(fidx[k]))
        V1 = g3[0][:, :3]
        V2 = g3[1][:, :3]
        V3 = g3[2][:, :3]
        coeff, geo = _face_geom(V1, V2, V3, with_geo=(k == 0))
        if k == 0:
            geo0 = jnp.pad(geo, ((0, 0), (0, 16 - 9)))
        coeffs.append(coeff)
        ones = _pad_rows(jnp.ones((Fs[k], 16), jnp.float32), Fpads[k])
        nfp = sc_scatter_add3(ones, *fidx[k], npads[k])
        inv_nfs.append(nf_inverse(nfp))

    def prep_W(W):
        cin = W.shape[1]
        return W.transpose(1, 0, 2).reshape(cin, K * G)

    def conv_layer(ff, k, W, b, third=True):
        Wp = prep_W(W)
        if third:
            Wp = Wp / 3.0
        msg = conv_matmul(ff, coeffs[k], Wp, b, Fs[k])
        p = sc_scatter_add3(msg, *fidx[k], npads[k])
        return combine_partials(p, inv_nfs[k], relu=True)

    W0 = jnp.pad(params['conv0_W'], ((0, 0), (0, 16 - 9), (0, 0)))
    feats = conv_layer(geo0, 0, W0, params['conv0_b'], third=False)

    skips = []
    for k in range(5):
        bp = params['blocks'][k]
        ff = None
        new = feats
        for W, b in zip(bp['Ws'], bp['bs']):
            ffnew = sc_gather_comb(new, list(fidx[k]), 'sum')
            ff = ffnew if ff is None else jnp.concatenate([ff, ffnew], axis=1)
            new = conv_layer(ff, k, W, b)
            feats = jnp.concatenate([feats, new], axis=1)
        feats = dense(feats, bp['Wout'], bp['bout'], relu=True)
        if k < 4:
            skips.append(feats)
            n, nn, a, b = vt_infos[k]
            apad = _pad_rows(a, npads[k + 1])
            bpad = _pad_rows(b, npads[k + 1])
            feats = sc_gather_comb(feats, [apad, bpad], 'max')

    for k in range(4):
        it = 4 - k
        n, nn, a, b = vt_infos[it - 1]
        vt_map = (jnp.arange(n, dtype=jnp.int32) * nn) // n
        up = sc_gather(feats, [_pad_rows(vt_map, npads[it - 1])])[0]
        feats = jnp.concatenate([skips[it - 1], up], axis=1)
        W, b = params['dec'][k]
        feats = dense(feats, W, b, relu=True)

    predW = jnp.pad(params['pred_W'], ((0, 0), (0, 128 - 13)))
    predb = jnp.pad(params['pred_b'], (0, 128 - 13))
    out = dense(feats, predW, predb, relu=False)
    return out[:ns[0], :13]
